# baseline (speedup 1.0000x reference)
import functools

import jax
import jax.numpy as jnp
from jax import lax
from jax.experimental import pallas as pl
from jax.experimental.pallas import tpu as pltpu
from jax.experimental.pallas import tpu_sc as plsc

N = 10000
E = 320000
D = 128
NC, NS = 2, 16
NW = NC * NS
EW = E // NW
B = 80
NB = EW // B
NWIN = 5
WB = NB // NWIN
WE = WB * B
RB = 400
NP = 10240
STRIPE = NP // NS
NCH = D // 16


def _pre_body(x_ref, wmsg_ref, a12_ref, wpass_ref, sig_ref, msgs_ref, ab_ref, pass_ref):
    x = x_ref[...]
    m = lax.dot_general(x, wmsg_ref[...], (((1,), (1,)), ((), ())),
                        preferred_element_type=jnp.float32)
    msgs_ref[...] = m
    ab_ref[...] = lax.dot_general(m, a12_ref[...], (((1,), (1,)), ((), ())),
                                  preferred_element_type=jnp.float32)
    pass_ref[...] = lax.dot_general(x, wpass_ref[...], (((1,), (1,)), ((), ())),
                                    preferred_element_type=jnp.float32) * sig_ref[0, 0]


def _post_body(acc_ref, den_ref, pass_ref, out_ref):
    num = acc_ref[0] + acc_ref[1]
    den = den_ref[0] + den_ref[1]
    merged = jnp.where(den > 0.0, num / den, 0.0)
    out_ref[...] = pass_ref[...] + jnp.maximum(merged, 0.0)


def _edge_body(src_hbm, dstr_hbm, alpha_hbm, beta_hbm, msgs_hbm, acc_hbm, den_hbm,
               src_v, dst_v, rows_v, aw_v, bw_v, w_v, acc_sh, den_sh,
               grs, gas, gbs, ssem, dsem):
    cid = lax.axis_index("c")
    sid = lax.axis_index("s")
    wid = cid * NS + sid
    eoff = wid * EW

    z16 = jnp.zeros((16,), jnp.float32)

    def zero_row(r, carry):
        for j in range(NCH):
            rows_v[0, r, pl.ds(j * 16, 16)] = z16
        return carry

    lax.fori_loop(0, B, zero_row, 0)
    for t in range(STRIPE // B):
        pltpu.sync_copy(rows_v.at[0], acc_sh.at[pl.ds(sid * STRIPE + t * B, B)])
    for sb in range(B // 16):
        w_v[0, pl.ds(sb * 16, 16)] = z16
    for t in range(STRIPE // B):
        pltpu.sync_copy(w_v.at[0], den_sh.at[pl.ds(sid * STRIPE + t * B, B)])
    plsc.subcore_barrier()

    def _gs(r, b):
        sidx = src_v.at[pl.ds(b * B, B)]
        pltpu.async_copy(msgs_hbm.at[sidx], rows_v.at[r], grs[r])
        pltpu.async_copy(alpha_hbm.at[sidx], aw_v.at[r], gas[r])
        pltpu.async_copy(beta_hbm.at[dst_v.at[b]], bw_v.at[r], gbs[r])

    def _gw(r, b):
        sidx = src_v.at[pl.ds(b * B, B)]
        pltpu.make_async_copy(msgs_hbm.at[sidx], rows_v.at[r], grs[r]).wait()
        pltpu.make_async_copy(alpha_hbm.at[sidx], aw_v.at[r], gas[r]).wait()
        pltpu.make_async_copy(beta_hbm.at[dst_v.at[b]], bw_v.at[r], gbs[r]).wait()

    def _compute(r, b):
        for sb in range(B // 16):
            av = aw_v[r, pl.ds(sb * 16, 16)]
            bv = bw_v[r, pl.ds(sb * 16, 16)]
            c = av + bv
            c = jnp.maximum(c, c * 0.01)
            w16 = jnp.exp(c)
            w_v[r, pl.ds(sb * 16, 16)] = w16
            for k in range(16):
                ws = jnp.broadcast_to(w16[k], (16,))
                e = sb * 16 + k
                for j in range(NCH):
                    rows_v[r, e, pl.ds(j * 16, 16)] = rows_v[r, e, pl.ds(j * 16, 16)] * ws

    def _ss(r, b):
        pltpu.async_copy(rows_v.at[r], acc_sh.at[dst_v.at[b]], ssem, add=True)
        pltpu.async_copy(w_v.at[r], den_sh.at[dst_v.at[b]], dsem, add=True)

    def _sw():
        pltpu.make_async_copy(rows_v.at[0], acc_sh.at[dst_v.at[0]], ssem).wait()
        pltpu.make_async_copy(w_v.at[0], den_sh.at[dst_v.at[0]], dsem).wait()

    def _slot(t, b):
        _gw(t, b)
        _compute(t, b)

        @pl.when(b > 0)
        def _():
            _sw()

        _ss(t, b)

        @pl.when(b + 2 < WB)
        def _():
            _gs((t + 2) % 3, b + 2)

    def triple_body(i, carry):
        b0 = 3 * i
        for t in range(3):
            _slot(t, b0 + t)
        return carry

    def window_body(wi, carry):
        pltpu.sync_copy(src_hbm.at[pl.ds(eoff + wi * WE, WE)], src_v)
        pltpu.sync_copy(dstr_hbm.at[wid, wi], dst_v)
        _gs(0, 0)
        _gs(1, 1)
        lax.fori_loop(0, (WB - 1) // 3, triple_body, 0)
        _slot(0, WB - 1)
        _sw()
        return carry

    lax.fori_loop(0, NWIN, window_body, 0)
    plsc.subcore_barrier()

    for t in range(STRIPE // 128):
        r0 = sid * STRIPE + t * 128
        pltpu.sync_copy(acc_sh.at[pl.ds(r0, 128)], acc_hbm.at[cid, pl.ds(r0, 128)])
    pltpu.sync_copy(den_sh.at[pl.ds(sid * STRIPE, STRIPE)],
                    den_hbm.at[cid, pl.ds(sid * STRIPE, STRIPE)])


_edge_kernel = functools.partial(
    pl.kernel,
    out_type=(
        jax.ShapeDtypeStruct((NC, NP, D), jnp.float32),
        jax.ShapeDtypeStruct((NC, NP), jnp.float32),
    ),
    mesh=plsc.VectorSubcoreMesh(core_axis_name="c", subcore_axis_name="s"),
    compiler_params=pltpu.CompilerParams(needs_layout_passes=False),
    scratch_types=[
        pltpu.VMEM((WE,), jnp.int32),
        pltpu.VMEM((WB, B), jnp.int32),
        pltpu.VMEM((3, B, D), jnp.float32),
        pltpu.VMEM((3, B), jnp.float32),
        pltpu.VMEM((3, B), jnp.float32),
        pltpu.VMEM((3, B), jnp.float32),
        pltpu.VMEM_SHARED((NP, D), jnp.float32),
        pltpu.VMEM_SHARED((NP,), jnp.float32),
        [pltpu.SemaphoreType.DMA] * 3,
        [pltpu.SemaphoreType.DMA] * 3,
        [pltpu.SemaphoreType.DMA] * 3,
        pltpu.SemaphoreType.DMA,
        pltpu.SemaphoreType.DMA,
    ],
)(_edge_body)


def kernel(in_states, edges, W_msg, a_att, W_pass, passthrough_coef):
    a12 = a_att[0, 0].reshape(2, D)
    sig = jax.nn.sigmoid(passthrough_coef).reshape(1, 1)
    grid = N // RB

    msgs, ab, pass_out = pl.pallas_call(
        _pre_body,
        grid=(grid,),
        in_specs=[
            pl.BlockSpec((RB, D), lambda i: (i, 0)),
            pl.BlockSpec((D, D), lambda i: (0, 0)),
            pl.BlockSpec((2, D), lambda i: (0, 0)),
            pl.BlockSpec((D, D), lambda i: (0, 0)),
            pl.BlockSpec(memory_space=pltpu.SMEM),
        ],
        out_specs=[
            pl.BlockSpec((RB, D), lambda i: (i, 0)),
            pl.BlockSpec((RB, 2), lambda i: (i, 0)),
            pl.BlockSpec((RB, D), lambda i: (i, 0)),
        ],
        out_shape=[
            jax.ShapeDtypeStruct((N, D), jnp.float32),
            jax.ShapeDtypeStruct((N, 2), jnp.float32),
            jax.ShapeDtypeStruct((N, D), jnp.float32),
        ],
    )(in_states, W_msg[0], a12, W_pass, sig)

    srcr = edges[0]
    dstr = edges[1].reshape(NW, NWIN, WB, B)
    acc, den = _edge_kernel(srcr, dstr, ab[:, 0], ab[:, 1], msgs)

    out = pl.pallas_call(
        _post_body,
        grid=(grid,),
        in_specs=[
            pl.BlockSpec((NC, RB, D), lambda i: (0, i, 0)),
            pl.BlockSpec((NC, RB, 1), lambda i: (0, i, 0)),
            pl.BlockSpec((RB, D), lambda i: (i, 0)),
        ],
        out_specs=pl.BlockSpec((RB, D), lambda i: (i, 0)),
        out_shape=jax.ShapeDtypeStruct((N, D), jnp.float32),
    )(acc, den.reshape(NC, NP, 1), pass_out)
    return out


# speedup vs baseline: 1.0038x; 1.0038x over previous
import functools

import jax
import jax.numpy as jnp
from jax import lax
from jax.experimental import pallas as pl
from jax.experimental.pallas import tpu as pltpu
from jax.experimental.pallas import tpu_sc as plsc

N = 10000
E = 320000
D = 128
NC, NS = 2, 16
NW = NC * NS
EW = E // NW
B = 80
NB = EW // B
NWIN = 5
WB = NB // NWIN
WE = WB * B
RB = 400
NP = 10240
STRIPE = NP // NS
NCH = D // 16


def _pre_body(x_ref, wmsg_ref, a12_ref, wpass_ref, sig_ref, msgs_ref, ab_ref, pass_ref):
    x = x_ref[...]
    m = lax.dot_general(x, wmsg_ref[...], (((1,), (1,)), ((), ())),
                        preferred_element_type=jnp.float32)
    msgs_ref[...] = m
    ab_ref[...] = lax.dot_general(m, a12_ref[...], (((1,), (1,)), ((), ())),
                                  preferred_element_type=jnp.float32)
    pass_ref[...] = lax.dot_general(x, wpass_ref[...], (((1,), (1,)), ((), ())),
                                    preferred_element_type=jnp.float32) * sig_ref[0, 0]


def _post_body(acc_ref, den_ref, pass_ref, out_ref):
    num = acc_ref[0] + acc_ref[1]
    den = den_ref[0] + den_ref[1]
    merged = jnp.where(den > 0.0, num / den, 0.0)
    out_ref[...] = pass_ref[...] + jnp.maximum(merged, 0.0)


def _edge_body(src_hbm, dstr_hbm, alpha_hbm, beta_hbm, msgs_hbm, acc_hbm, den_hbm,
               src_v, dst_v, rows_v, aw_v, bw_v, w_v, acc_sh, den_sh,
               grs, gas, gbs, ssem, dsem):
    cid = lax.axis_index("c")
    sid = lax.axis_index("s")
    wid = cid * NS + sid
    eoff = wid * EW

    z16 = jnp.zeros((16,), jnp.float32)

    def zero_row(r, carry):
        for j in range(NCH):
            rows_v[0, r, pl.ds(j * 16, 16)] = z16
        return carry

    lax.fori_loop(0, B, zero_row, 0)
    for t in range(STRIPE // B):
        pltpu.sync_copy(rows_v.at[0], acc_sh.at[pl.ds(sid * STRIPE + t * B, B)])
    for sb in range(B // 16):
        w_v[0, pl.ds(sb * 16, 16)] = z16
    for t in range(STRIPE // B):
        pltpu.sync_copy(w_v.at[0], den_sh.at[pl.ds(sid * STRIPE + t * B, B)])
    plsc.subcore_barrier()

    def _gs(r, b):
        sidx = src_v.at[pl.ds(b * B, B)]
        pltpu.async_copy(msgs_hbm.at[sidx], rows_v.at[r], grs[r])
        pltpu.async_copy(alpha_hbm.at[sidx], aw_v.at[r], gas[r])
        pltpu.async_copy(beta_hbm.at[dst_v.at[b]], bw_v.at[r], gbs[r])

    def _gw(r, b):
        sidx = src_v.at[pl.ds(b * B, B)]
        pltpu.make_async_copy(msgs_hbm.at[sidx], rows_v.at[r], grs[r]).wait()
        pltpu.make_async_copy(alpha_hbm.at[sidx], aw_v.at[r], gas[r]).wait()
        pltpu.make_async_copy(beta_hbm.at[dst_v.at[b]], bw_v.at[r], gbs[r]).wait()

    def _compute(r, b):
        for sb in range(B // 16):
            av = aw_v[r, pl.ds(sb * 16, 16)]
            bv = bw_v[r, pl.ds(sb * 16, 16)]
            c = av + bv
            c = jnp.maximum(c, c * 0.01)
            w16 = jnp.exp(c)
            w_v[r, pl.ds(sb * 16, 16)] = w16
            for k in range(16):
                ws = jnp.broadcast_to(w16[k], (16,))
                e = sb * 16 + k
                for j in range(NCH):
                    rows_v[r, e, pl.ds(j * 16, 16)] = rows_v[r, e, pl.ds(j * 16, 16)] * ws

    def _ss(t, b):
        pltpu.async_copy(rows_v.at[t], acc_sh.at[dst_v.at[b]], ssem[t], add=True)
        pltpu.async_copy(w_v.at[t], den_sh.at[dst_v.at[b]], dsem[t], add=True)

    def _sw(t):
        pltpu.make_async_copy(rows_v.at[0], acc_sh.at[dst_v.at[0]], ssem[t]).wait()
        pltpu.make_async_copy(w_v.at[0], den_sh.at[dst_v.at[0]], dsem[t]).wait()

    def _slot(t, b):
        _gw(t, b)
        _compute(t, b)
        _ss(t, b)

        @pl.when(b + 2 < WB)
        def _():
            @pl.when(b > 0)
            def _():
                _sw((t + 2) % 3)

            _gs((t + 2) % 3, b + 2)

    def triple_body(i, carry):
        b0 = 3 * i
        for t in range(3):
            _slot(t, b0 + t)
        return carry

    def window_body(wi, carry):
        pltpu.sync_copy(src_hbm.at[pl.ds(eoff + wi * WE, WE)], src_v)
        pltpu.sync_copy(dstr_hbm.at[wid, wi], dst_v)
        _gs(0, 0)
        _gs(1, 1)
        lax.fori_loop(0, (WB - 1) // 3, triple_body, 0)
        _slot(0, WB - 1)
        for t in range(3):
            _sw(t)
        return carry

    lax.fori_loop(0, NWIN, window_body, 0)
    plsc.subcore_barrier()

    for t in range(STRIPE // 128):
        r0 = sid * STRIPE + t * 128
        pltpu.sync_copy(acc_sh.at[pl.ds(r0, 128)], acc_hbm.at[cid, pl.ds(r0, 128)])
    pltpu.sync_copy(den_sh.at[pl.ds(sid * STRIPE, STRIPE)],
                    den_hbm.at[cid, pl.ds(sid * STRIPE, STRIPE)])


_edge_kernel = functools.partial(
    pl.kernel,
    out_type=(
        jax.ShapeDtypeStruct((NC, NP, D), jnp.float32),
        jax.ShapeDtypeStruct((NC, NP), jnp.float32),
    ),
    mesh=plsc.VectorSubcoreMesh(core_axis_name="c", subcore_axis_name="s"),
    compiler_params=pltpu.CompilerParams(needs_layout_passes=False),
    scratch_types=[
        pltpu.VMEM((WE,), jnp.int32),
        pltpu.VMEM((WB, B), jnp.int32),
        pltpu.VMEM((3, B, D), jnp.float32),
        pltpu.VMEM((3, B), jnp.float32),
        pltpu.VMEM((3, B), jnp.float32),
        pltpu.VMEM((3, B), jnp.float32),
        pltpu.VMEM_SHARED((NP, D), jnp.float32),
        pltpu.VMEM_SHARED((NP,), jnp.float32),
        [pltpu.SemaphoreType.DMA] * 3,
        [pltpu.SemaphoreType.DMA] * 3,
        [pltpu.SemaphoreType.DMA] * 3,
        [pltpu.SemaphoreType.DMA] * 3,
        [pltpu.SemaphoreType.DMA] * 3,
    ],
)(_edge_body)


def kernel(in_states, edges, W_msg, a_att, W_pass, passthrough_coef):
    a12 = a_att[0, 0].reshape(2, D)
    sig = jax.nn.sigmoid(passthrough_coef).reshape(1, 1)
    grid = N // RB

    msgs, ab, pass_out = pl.pallas_call(
        _pre_body,
        grid=(grid,),
        in_specs=[
            pl.BlockSpec((RB, D), lambda i: (i, 0)),
            pl.BlockSpec((D, D), lambda i: (0, 0)),
            pl.BlockSpec((2, D), lambda i: (0, 0)),
            pl.BlockSpec((D, D), lambda i: (0, 0)),
            pl.BlockSpec(memory_space=pltpu.SMEM),
        ],
        out_specs=[
            pl.BlockSpec((RB, D), lambda i: (i, 0)),
            pl.BlockSpec((RB, 2), lambda i: (i, 0)),
            pl.BlockSpec((RB, D), lambda i: (i, 0)),
        ],
        out_shape=[
            jax.ShapeDtypeStruct((N, D), jnp.float32),
            jax.ShapeDtypeStruct((N, 2), jnp.float32),
            jax.ShapeDtypeStruct((N, D), jnp.float32),
        ],
    )(in_states, W_msg[0], a12, W_pass, sig)

    srcr = edges[0]
    dstr = edges[1].reshape(NW, NWIN, WB, B)
    acc, den = _edge_kernel(srcr, dstr, ab[:, 0], ab[:, 1], msgs)

    out = pl.pallas_call(
        _post_body,
        grid=(grid,),
        in_specs=[
            pl.BlockSpec((NC, RB, D), lambda i: (0, i, 0)),
            pl.BlockSpec((NC, RB, 1), lambda i: (0, i, 0)),
            pl.BlockSpec((RB, D), lambda i: (i, 0)),
        ],
        out_specs=pl.BlockSpec((RB, D), lambda i: (i, 0)),
        out_shape=jax.ShapeDtypeStruct((N, D), jnp.float32),
    )(acc, den.reshape(NC, NP, 1), pass_out)
    return out


# speedup vs baseline: 1.0610x; 1.0570x over previous
import functools

import jax
import jax.numpy as jnp
from jax import lax
from jax.experimental import pallas as pl
from jax.experimental.pallas import tpu as pltpu
from jax.experimental.pallas import tpu_sc as plsc

N = 10000
E = 320000
D = 128
NC, NS = 2, 16
NW = NC * NS
EW = E // NW
B = 80
NB = EW // B
NWIN = 5
WB = NB // NWIN
WE = WB * B
RB = 1000
NP = 10240
STRIPE = NP // NS
NCH = D // 16


def _pre_body(x_ref, wmsg_ref, a12_ref, wpass_ref, sig_ref, msgs_ref, ab_ref, pass_ref):
    x = x_ref[...]
    m = lax.dot_general(x, wmsg_ref[...], (((1,), (1,)), ((), ())),
                        preferred_element_type=jnp.float32)
    msgs_ref[...] = m
    ab_ref[...] = lax.dot_general(m, a12_ref[...], (((1,), (1,)), ((), ())),
                                  preferred_element_type=jnp.float32)
    pass_ref[...] = lax.dot_general(x, wpass_ref[...], (((1,), (1,)), ((), ())),
                                    preferred_element_type=jnp.float32) * sig_ref[0, 0]


def _post_body(acc_ref, den_ref, pass_ref, out_ref):
    num = acc_ref[0] + acc_ref[1]
    den = den_ref[0] + den_ref[1]
    merged = jnp.where(den > 0.0, num / den, 0.0)
    out_ref[...] = pass_ref[...] + jnp.maximum(merged, 0.0)


def _edge_body(src_hbm, dstr_hbm, alpha_hbm, beta_hbm, msgs_hbm, acc_hbm, den_hbm,
               src_v, dst_v, rows_v, aw_v, bw_v, w_v, acc_sh, den_sh,
               grs, gas, gbs, ssem, dsem):
    cid = lax.axis_index("c")
    sid = lax.axis_index("s")
    wid = cid * NS + sid
    eoff = wid * EW

    z16 = jnp.zeros((16,), jnp.float32)

    def zero_row(r, carry):
        for j in range(NCH):
            rows_v[0, r, pl.ds(j * 16, 16)] = z16
        return carry

    lax.fori_loop(0, B, zero_row, 0)
    for t in range(STRIPE // B):
        pltpu.sync_copy(rows_v.at[0], acc_sh.at[pl.ds(sid * STRIPE + t * B, B)])
    for sb in range(B // 16):
        w_v[0, pl.ds(sb * 16, 16)] = z16
    for t in range(STRIPE // B):
        pltpu.sync_copy(w_v.at[0], den_sh.at[pl.ds(sid * STRIPE + t * B, B)])
    plsc.subcore_barrier()

    def _gs(r, b):
        sidx = src_v.at[pl.ds(b * B, B)]
        pltpu.async_copy(msgs_hbm.at[sidx], rows_v.at[r], grs[r])
        pltpu.async_copy(alpha_hbm.at[sidx], aw_v.at[r], gas[r])
        pltpu.async_copy(beta_hbm.at[dst_v.at[b]], bw_v.at[r], gbs[r])

    def _gw(r, b):
        sidx = src_v.at[pl.ds(b * B, B)]
        pltpu.make_async_copy(msgs_hbm.at[sidx], rows_v.at[r], grs[r]).wait()
        pltpu.make_async_copy(alpha_hbm.at[sidx], aw_v.at[r], gas[r]).wait()
        pltpu.make_async_copy(beta_hbm.at[dst_v.at[b]], bw_v.at[r], gbs[r]).wait()

    def _compute(r, b):
        for sb in range(B // 16):
            av = aw_v[r, pl.ds(sb * 16, 16)]
            bv = bw_v[r, pl.ds(sb * 16, 16)]
            c = av + bv
            c = jnp.maximum(c, c * 0.01)
            w16 = jnp.exp(c)
            w_v[r, pl.ds(sb * 16, 16)] = w16
            for k in range(16):
                ws = jnp.broadcast_to(w16[k], (16,))
                e = sb * 16 + k
                for j in range(NCH):
                    rows_v[r, e, pl.ds(j * 16, 16)] = rows_v[r, e, pl.ds(j * 16, 16)] * ws

    def _ss(t, b):
        pltpu.async_copy(rows_v.at[t], acc_sh.at[dst_v.at[b]], ssem[t], add=True)
        pltpu.async_copy(w_v.at[t], den_sh.at[dst_v.at[b]], dsem[t], add=True)

    def _sw(t):
        pltpu.make_async_copy(rows_v.at[0], acc_sh.at[dst_v.at[0]], ssem[t]).wait()
        pltpu.make_async_copy(w_v.at[0], den_sh.at[dst_v.at[0]], dsem[t]).wait()

    def _slot(t, b):
        _gw(t, b)
        _compute(t, b)
        _ss(t, b)

        @pl.when(b + 2 < WB)
        def _():
            @pl.when(b > 0)
            def _():
                _sw((t + 2) % 3)

            _gs((t + 2) % 3, b + 2)

    def triple_body(i, carry):
        b0 = 3 * i
        for t in range(3):
            _slot(t, b0 + t)
        return carry

    def window_body(wi, carry):
        pltpu.sync_copy(src_hbm.at[pl.ds(eoff + wi * WE, WE)], src_v)
        pltpu.sync_copy(dstr_hbm.at[wid, wi], dst_v)
        _gs(0, 0)
        _gs(1, 1)
        lax.fori_loop(0, (WB - 1) // 3, triple_body, 0)
        _slot(0, WB - 1)
        for t in range(3):
            _sw(t)
        return carry

    lax.fori_loop(0, NWIN, window_body, 0)
    plsc.subcore_barrier()

    for t in range(STRIPE // 128):
        r0 = sid * STRIPE + t * 128
        pltpu.sync_copy(acc_sh.at[pl.ds(r0, 128)], acc_hbm.at[cid, pl.ds(r0, 128)])
    pltpu.sync_copy(den_sh.at[pl.ds(sid * STRIPE, STRIPE)],
                    den_hbm.at[cid, pl.ds(sid * STRIPE, STRIPE)])


_edge_kernel = functools.partial(
    pl.kernel,
    out_type=(
        jax.ShapeDtypeStruct((NC, NP, D), jnp.float32),
        jax.ShapeDtypeStruct((NC, NP), jnp.float32),
    ),
    mesh=plsc.VectorSubcoreMesh(core_axis_name="c", subcore_axis_name="s"),
    compiler_params=pltpu.CompilerParams(needs_layout_passes=False),
    scratch_types=[
        pltpu.VMEM((WE,), jnp.int32),
        pltpu.VMEM((WB, B), jnp.int32),
        pltpu.VMEM((3, B, D), jnp.float32),
        pltpu.VMEM((3, B), jnp.float32),
        pltpu.VMEM((3, B), jnp.float32),
        pltpu.VMEM((3, B), jnp.float32),
        pltpu.VMEM_SHARED((NP, D), jnp.float32),
        pltpu.VMEM_SHARED((NP,), jnp.float32),
        [pltpu.SemaphoreType.DMA] * 3,
        [pltpu.SemaphoreType.DMA] * 3,
        [pltpu.SemaphoreType.DMA] * 3,
        [pltpu.SemaphoreType.DMA] * 3,
        [pltpu.SemaphoreType.DMA] * 3,
    ],
)(_edge_body)


def kernel(in_states, edges, W_msg, a_att, W_pass, passthrough_coef):
    a12 = a_att[0, 0].reshape(2, D)
    sig = jax.nn.sigmoid(passthrough_coef).reshape(1, 1)
    grid = N // RB

    msgs, ab, pass_out = pl.pallas_call(
        _pre_body,
        grid=(grid,),
        in_specs=[
            pl.BlockSpec((RB, D), lambda i: (i, 0)),
            pl.BlockSpec((D, D), lambda i: (0, 0)),
            pl.BlockSpec((2, D), lambda i: (0, 0)),
            pl.BlockSpec((D, D), lambda i: (0, 0)),
            pl.BlockSpec(memory_space=pltpu.SMEM),
        ],
        out_specs=[
            pl.BlockSpec((RB, D), lambda i: (i, 0)),
            pl.BlockSpec((RB, 2), lambda i: (i, 0)),
            pl.BlockSpec((RB, D), lambda i: (i, 0)),
        ],
        out_shape=[
            jax.ShapeDtypeStruct((N, D), jnp.float32),
            jax.ShapeDtypeStruct((N, 2), jnp.float32),
            jax.ShapeDtypeStruct((N, D), jnp.float32),
        ],
    )(in_states, W_msg[0], a12, W_pass, sig)

    srcr = edges[0]
    dstr = edges[1].reshape(NW, NWIN, WB, B)
    acc, den = _edge_kernel(srcr, dstr, ab[:, 0], ab[:, 1], msgs)

    out = pl.pallas_call(
        _post_body,
        grid=(grid,),
        in_specs=[
            pl.BlockSpec((NC, RB, D), lambda i: (0, i, 0)),
            pl.BlockSpec((NC, RB, 1), lambda i: (0, i, 0)),
            pl.BlockSpec((RB, D), lambda i: (i, 0)),
        ],
        out_specs=pl.BlockSpec((RB, D), lambda i: (i, 0)),
        out_shape=jax.ShapeDtypeStruct((N, D), jnp.float32),
    )(acc, den.reshape(NC, NP, 1), pass_out)
    return out
